# baseline (speedup 1.0000x reference)
import functools

import jax
import jax.numpy as jnp
from jax import lax
from jax.experimental import pallas as pl
from jax.experimental.pallas import tpu as pltpu
from jax.experimental.pallas import tpu_sc as plsc

N_ROWS = 16384
N_COLS = 16384
K = 163
KP = 176
KH = KP // 2
BATCH = 256
NC = 2
NS = 16
NW = NC * NS
ROWS_PER_W = N_ROWS // NW
LANES = 16
VB = BATCH // LANES
CHUNK = 32
N_CHUNKS = ROWS_PER_W // CHUNK


def _spmv_sc(xT, vals, cols):
    mesh = plsc.VectorSubcoreMesh(core_axis_name="c", subcore_axis_name="s")

    @functools.partial(
        pl.kernel,
        mesh=mesh,
        out_type=jax.ShapeDtypeStruct((N_ROWS, BATCH), jnp.float32),
        scratch_types=[
            pltpu.VMEM((CHUNK, 2, KH), jnp.int32),
            pltpu.VMEM((CHUNK, KP), jnp.float32),
            pltpu.VMEM((KP, BATCH), jnp.float32),
            pltpu.VMEM((KP, BATCH), jnp.float32),
            pltpu.VMEM((CHUNK, BATCH), jnp.float32),
            pltpu.SemaphoreType.DMA,
            pltpu.SemaphoreType.DMA,
        ],
    )
    def k(xT_hbm, vals_hbm, cols_hbm, out_hbm, idx_v, vals_v, rows0, rows1,
          out_v, sem0, sem1):
        wid = lax.axis_index("s") * NC + lax.axis_index("c")
        row0 = wid * ROWS_PER_W

        def gather_start(r, dst, sem):
            pltpu.async_copy(xT_hbm.at[idx_v.at[r].at[0]],
                             dst.at[pl.ds(0, KH)], sem)
            pltpu.async_copy(xT_hbm.at[idx_v.at[r].at[1]],
                             dst.at[pl.ds(KH, KH)], sem)

        def gather_wait(dst, sem):
            pltpu.make_async_copy(xT_hbm.at[idx_v.at[0].at[0]],
                                  dst.at[pl.ds(0, KH)], sem).wait()
            pltpu.make_async_copy(xT_hbm.at[idx_v.at[0].at[1]],
                                  dst.at[pl.ds(KH, KH)], sem).wait()

        def compute(r, src):
            HB = VB // 2

            for h in range(2):
                def j_body(jb, accs, h=h):
                    j0 = pl.multiple_of(jb * LANES, LANES)
                    vblock = vals_v[r, pl.ds(j0, LANES)]
                    for t in range(LANES):
                        vv = jnp.full((LANES,), vblock[t], jnp.float32)
                        accs = tuple(
                            accs[k] + vv * src[
                                j0 + t, pl.ds((h * HB + k) * LANES, LANES)]
                            for k in range(HB))
                    return accs

                accs = lax.fori_loop(
                    0, KP // LANES, j_body,
                    tuple(jnp.zeros((LANES,), jnp.float32)
                          for _ in range(HB)))
                for k in range(HB):
                    out_v[r, pl.ds((h * HB + k) * LANES, LANES)] = accs[k]

        def chunk_body(c, carry):
            base = row0 + c * CHUNK
            pltpu.sync_copy(cols_hbm.at[pl.ds(base, CHUNK)], idx_v)
            pltpu.sync_copy(vals_hbm.at[pl.ds(base, CHUNK)], vals_v)
            gather_start(0, rows0, sem0)

            def pair_body(i, carry):
                r2 = i * 2
                gather_start(r2 + 1, rows1, sem1)
                gather_wait(rows0, sem0)
                compute(r2, rows0)

                @pl.when(r2 + 2 < CHUNK)
                def _():
                    gather_start(r2 + 2, rows0, sem0)

                gather_wait(rows1, sem1)
                compute(r2 + 1, rows1)
                return carry

            lax.fori_loop(0, CHUNK // 2, pair_body, 0)
            pltpu.sync_copy(out_v, out_hbm.at[pl.ds(base, CHUNK)])
            return carry

        lax.fori_loop(0, N_CHUNKS, chunk_body, 0)

    return k(xT, vals, cols)


@jax.jit
def kernel(x, c_0, c_1, c_2):
    del c_2
    xT = x.T
    vals = jnp.zeros((N_ROWS, KP), jnp.float32)
    vals = vals.at[:, :K].set(c_0.reshape(N_ROWS, K))
    cols = jnp.zeros((N_ROWS, KP), jnp.int32)
    cols = cols.at[:, :K].set(c_1.reshape(N_ROWS, K))
    cols = cols.reshape(N_ROWS, 2, KH)
    yT = _spmv_sc(xT, vals, cols)
    return yT.T


# speedup vs baseline: 1.0019x; 1.0019x over previous
import functools

import jax
import jax.numpy as jnp
from jax import lax
from jax.experimental import pallas as pl
from jax.experimental.pallas import tpu as pltpu
from jax.experimental.pallas import tpu_sc as plsc

N_ROWS = 16384
N_COLS = 16384
K = 163
KP = 176
KH = KP // 2
BATCH = 256
NC = 2
NS = 16
NW = NC * NS
ROWS_PER_W = N_ROWS // NW
LANES = 16
VB = BATCH // LANES
CHUNK = 32
N_CHUNKS = ROWS_PER_W // CHUNK


def _spmv_sc(xT, vals, cols):
    mesh = plsc.VectorSubcoreMesh(core_axis_name="c", subcore_axis_name="s")

    @functools.partial(
        pl.kernel,
        mesh=mesh,
        out_type=jax.ShapeDtypeStruct((N_ROWS, BATCH), jnp.float32),
        scratch_types=[
            pltpu.VMEM((CHUNK, KP), jnp.int32),
            pltpu.VMEM((CHUNK, KP), jnp.float32),
            pltpu.VMEM((KP, BATCH), jnp.float32),
            pltpu.VMEM((KP, BATCH), jnp.float32),
            pltpu.VMEM((CHUNK, BATCH), jnp.float32),
            pltpu.SemaphoreType.DMA,
            pltpu.SemaphoreType.DMA,
        ],
    )
    def k(xT_hbm, vals_hbm, cols_hbm, out_hbm, idx_v, vals_v, rows0, rows1,
          out_v, sem0, sem1):
        wid = lax.axis_index("s") * NC + lax.axis_index("c")
        row0 = wid * ROWS_PER_W

        def gather_start(r, dst, sem):
            for q in range(KP // LANES):
                pltpu.async_copy(
                    xT_hbm.at[idx_v.at[r, pl.ds(q * LANES, LANES)]],
                    dst.at[pl.ds(q * LANES, LANES)], sem)

        def gather_wait(dst, sem):
            for q in range(KP // LANES):
                pltpu.make_async_copy(
                    xT_hbm.at[idx_v.at[0, pl.ds(q * LANES, LANES)]],
                    dst.at[pl.ds(q * LANES, LANES)], sem).wait()

        def compute(r, src):
            HB = VB // 2

            for h in range(2):
                def j_body(jb, accs, h=h):
                    j0 = pl.multiple_of(jb * LANES, LANES)
                    vblock = vals_v[r, pl.ds(j0, LANES)]
                    for t in range(LANES):
                        vv = jnp.full((LANES,), vblock[t], jnp.float32)
                        accs = tuple(
                            accs[k] + vv * src[
                                j0 + t, pl.ds((h * HB + k) * LANES, LANES)]
                            for k in range(HB))
                    return accs

                accs = lax.fori_loop(
                    0, KP // LANES, j_body,
                    tuple(jnp.zeros((LANES,), jnp.float32)
                          for _ in range(HB)))
                for k in range(HB):
                    out_v[r, pl.ds((h * HB + k) * LANES, LANES)] = accs[k]

        def chunk_body(c, carry):
            base = row0 + c * CHUNK
            pltpu.sync_copy(cols_hbm.at[pl.ds(base, CHUNK)], idx_v)
            pltpu.sync_copy(vals_hbm.at[pl.ds(base, CHUNK)], vals_v)
            gather_start(0, rows0, sem0)

            def pair_body(i, carry):
                r2 = i * 2
                gather_start(r2 + 1, rows1, sem1)
                gather_wait(rows0, sem0)
                compute(r2, rows0)

                @pl.when(r2 + 2 < CHUNK)
                def _():
                    gather_start(r2 + 2, rows0, sem0)

                gather_wait(rows1, sem1)
                compute(r2 + 1, rows1)
                return carry

            lax.fori_loop(0, CHUNK // 2, pair_body, 0)
            pltpu.sync_copy(out_v, out_hbm.at[pl.ds(base, CHUNK)])
            return carry

        lax.fori_loop(0, N_CHUNKS, chunk_body, 0)

    return k(xT, vals, cols)


@jax.jit
def kernel(x, c_0, c_1, c_2):
    del c_2
    xT = x.T
    vals = jnp.zeros((N_ROWS, KP), jnp.float32)
    vals = vals.at[:, :K].set(c_0.reshape(N_ROWS, K))
    cols = jnp.zeros((N_ROWS, KP), jnp.int32)
    cols = cols.at[:, :K].set(c_1.reshape(N_ROWS, K))
    yT = _spmv_sc(xT, vals, cols)
    return yT.T


# speedup vs baseline: 1.0518x; 1.0498x over previous
import functools

import jax
import jax.numpy as jnp
from jax import lax
from jax.experimental import pallas as pl
from jax.experimental.pallas import tpu as pltpu
from jax.experimental.pallas import tpu_sc as plsc

N_ROWS = 16384
N_COLS = 16384
K = 163
KP = 176
KH = KP // 2
BATCH = 256
NC = 2
NS = 16
NW = NC * NS
ROWS_PER_W = N_ROWS // NW
LANES = 16
VB = BATCH // LANES
CHUNK = 32
N_CHUNKS = ROWS_PER_W // CHUNK


def _spmv_sc(xT, vals, cols):
    mesh = plsc.VectorSubcoreMesh(core_axis_name="c", subcore_axis_name="s")

    @functools.partial(
        pl.kernel,
        mesh=mesh,
        out_type=jax.ShapeDtypeStruct((N_ROWS, BATCH), jnp.float32),
        compiler_params=pltpu.CompilerParams(use_tc_tiling_on_sc=False,
                                             needs_layout_passes=False),
        scratch_types=[
            pltpu.VMEM((CHUNK, KP), jnp.int32),
            pltpu.VMEM((CHUNK, KP), jnp.float32),
            pltpu.VMEM((KP, BATCH), jnp.bfloat16),
            pltpu.VMEM((KP, BATCH), jnp.bfloat16),
            pltpu.VMEM((CHUNK, BATCH), jnp.float32),
            pltpu.SemaphoreType.DMA,
            pltpu.SemaphoreType.DMA,
        ],
    )
    def k(xT_hbm, vals_hbm, cols_hbm, out_hbm, idx_v, vals_v, rows0, rows1,
          out_v, sem0, sem1):
        wid = lax.axis_index("s") * NC + lax.axis_index("c")
        row0 = wid * ROWS_PER_W

        def gather_start(r, dst, sem):
            for q in range(KP // LANES):
                pltpu.async_copy(
                    xT_hbm.at[idx_v.at[r, pl.ds(q * LANES, LANES)]],
                    dst.at[pl.ds(q * LANES, LANES)], sem)

        def gather_wait(dst, sem):
            for q in range(KP // LANES):
                pltpu.make_async_copy(
                    xT_hbm.at[idx_v.at[0, pl.ds(q * LANES, LANES)]],
                    dst.at[pl.ds(q * LANES, LANES)], sem).wait()

        def compute(r, src):
            HB = VB // 2

            for h in range(2):
                def j_body(jb, accs, h=h):
                    j0 = pl.multiple_of(jb * LANES, LANES)
                    vblock = vals_v[r, pl.ds(j0, LANES)]
                    for t in range(LANES):
                        vv = jnp.full((LANES,), vblock[t], jnp.float32)
                        new = list(accs)
                        for u in range(HB // 2):
                            ab = src[j0 + t,
                                     pl.ds(h * (HB * LANES) + u * 2 * LANES,
                                           2 * LANES)]
                            a, b = plsc.unpack(
                                ab, format=plsc.PackFormat.INTERLEAVED)
                            new[2 * u] = new[2 * u] + vv * a
                            new[2 * u + 1] = new[2 * u + 1] + vv * b
                        accs = tuple(new)
                    return accs

                accs = lax.fori_loop(
                    0, KP // LANES, j_body,
                    tuple(jnp.zeros((LANES,), jnp.float32)
                          for _ in range(HB)))
                for k in range(HB):
                    out_v[r, pl.ds((h * HB + k) * LANES, LANES)] = accs[k]

        def chunk_body(c, carry):
            base = row0 + c * CHUNK
            pltpu.sync_copy(cols_hbm.at[pl.ds(base, CHUNK)], idx_v)
            pltpu.sync_copy(vals_hbm.at[pl.ds(base, CHUNK)], vals_v)
            gather_start(0, rows0, sem0)

            def pair_body(i, carry):
                r2 = i * 2
                gather_start(r2 + 1, rows1, sem1)
                gather_wait(rows0, sem0)
                compute(r2, rows0)

                @pl.when(r2 + 2 < CHUNK)
                def _():
                    gather_start(r2 + 2, rows0, sem0)

                gather_wait(rows1, sem1)
                compute(r2 + 1, rows1)
                return carry

            lax.fori_loop(0, CHUNK // 2, pair_body, 0)
            pltpu.sync_copy(out_v, out_hbm.at[pl.ds(base, CHUNK)])
            return carry

        lax.fori_loop(0, N_CHUNKS, chunk_body, 0)

    return k(xT, vals, cols)


def _interleave_perm():
    import numpy as np
    p = np.empty((BATCH,), np.int32)
    for k in range(BATCH // 32):
        for t in range(16):
            p[32 * k + 2 * t] = 32 * k + t
            p[32 * k + 2 * t + 1] = 32 * k + 16 + t
    return p


_PERM = _interleave_perm()


@jax.jit
def kernel(x, c_0, c_1, c_2):
    del c_2
    xT = x[jnp.asarray(_PERM)].T.astype(jnp.bfloat16)
    vals = jnp.zeros((N_ROWS, KP), jnp.float32)
    vals = vals.at[:, :K].set(c_0.reshape(N_ROWS, K))
    cols = jnp.zeros((N_ROWS, KP), jnp.int32)
    cols = cols.at[:, :K].set(c_1.reshape(N_ROWS, K))
    yT = _spmv_sc(xT, vals, cols)
    return yT.T


# speedup vs baseline: 1.6489x; 1.5676x over previous
import functools

import jax
import jax.numpy as jnp
import numpy as np
from jax import lax
from jax.experimental import pallas as pl
from jax.experimental.pallas import tpu as pltpu
from jax.experimental.pallas import tpu_sc as plsc

N_ROWS = 16384
N_COLS = 16384
K = 163
KP = 176
HALF_COLS = N_COLS // 2
BATCH = 256
NC = 2
NS = 16
ROWS_PER_W = N_ROWS // NS
LANES = 16
VB = BATCH // LANES
CHUNK = 16
N_CHUNKS = ROWS_PER_W // CHUNK
LOAD_ROWS = HALF_COLS // NS
LOAD_STEP = 128


def _spmv_sc(table, vals, cols):
    mesh = plsc.VectorSubcoreMesh(core_axis_name="c", subcore_axis_name="s")

    @functools.partial(
        pl.kernel,
        mesh=mesh,
        out_type=jax.ShapeDtypeStruct((NC, N_ROWS, BATCH), jnp.float32),
        compiler_params=pltpu.CompilerParams(use_tc_tiling_on_sc=False,
                                             needs_layout_passes=False),
        scratch_types=[
            pltpu.VMEM_SHARED((HALF_COLS, BATCH // 2), jnp.int32),
            pltpu.VMEM((CHUNK, KP), jnp.int32),
            pltpu.VMEM((CHUNK, KP), jnp.float32),
            pltpu.VMEM((KP, BATCH // 2), jnp.int32),
            pltpu.VMEM((KP, BATCH // 2), jnp.int32),
            pltpu.VMEM((CHUNK, BATCH), jnp.float32),
            pltpu.SemaphoreType.DMA,
            pltpu.SemaphoreType.DMA,
        ],
    )
    def k(table_hbm, vals_hbm, cols_hbm, out_hbm, table_sp, idx_v, vals_v,
          rows0, rows1, out_v, sem0, sem1):
        ci = lax.axis_index("c")
        si = lax.axis_index("s")

        def stage_body(b, carry):
            src_row = ci * HALF_COLS + si * LOAD_ROWS + b * LOAD_STEP
            dst_row = si * LOAD_ROWS + b * LOAD_STEP
            pltpu.sync_copy(table_hbm.at[pl.ds(src_row, LOAD_STEP)],
                            rows0.at[pl.ds(0, LOAD_STEP)])
            pltpu.sync_copy(rows0.at[pl.ds(0, LOAD_STEP)],
                            table_sp.at[pl.ds(dst_row, LOAD_STEP)])
            return carry

        lax.fori_loop(0, LOAD_ROWS // LOAD_STEP, stage_body, 0)
        plsc.subcore_barrier()

        row0 = si * ROWS_PER_W

        def gather_start(r, dst, sem):
            for q in range(KP // LANES):
                iv = idx_v[r, pl.ds(q * LANES, LANES)]
                pltpu.async_copy(
                    table_sp.at[iv],
                    dst.at[pl.ds(q * LANES, LANES)], sem)

        def gather_wait(dst, sem):
            zv = jnp.zeros((LANES,), jnp.int32)
            for q in range(KP // LANES):
                pltpu.make_async_copy(
                    table_sp.at[zv],
                    dst.at[pl.ds(q * LANES, LANES)], sem).wait()

        def compute(r, src):
            HB = VB // 2

            for h in range(2):
                def j_body(jb, accs, h=h):
                    j0 = pl.multiple_of(jb * LANES, LANES)
                    vblock = vals_v[r, pl.ds(j0, LANES)]
                    for t in range(LANES):
                        vv = jnp.full((LANES,), vblock[t], jnp.float32)
                        new = list(accs)
                        for u in range(HB // 2):
                            ld = src[j0 + t,
                                     pl.ds(h * (HB * LANES) // 2 + u * LANES,
                                           LANES)]
                            ab = plsc.bitcast(ld, jnp.bfloat16)
                            a, b = plsc.unpack(
                                ab, format=plsc.PackFormat.INTERLEAVED)
                            new[2 * u] = new[2 * u] + vv * a
                            new[2 * u + 1] = new[2 * u + 1] + vv * b
                        accs = tuple(new)
                    return accs

                accs = lax.fori_loop(
                    0, KP // LANES, j_body,
                    tuple(jnp.zeros((LANES,), jnp.float32)
                          for _ in range(HB)))
                for k2 in range(HB):
                    out_v[r, pl.ds((h * HB + k2) * LANES, LANES)] = accs[k2]

        def chunk_body(c, carry):
            base = row0 + c * CHUNK
            pltpu.sync_copy(cols_hbm.at[ci].at[pl.ds(base, CHUNK)], idx_v)
            pltpu.sync_copy(vals_hbm.at[ci].at[pl.ds(base, CHUNK)], vals_v)
            gather_start(0, rows0, sem0)

            def pair_body(i, carry):
                r2 = i * 2
                gather_start(r2 + 1, rows1, sem1)
                gather_wait(rows0, sem0)
                compute(r2, rows0)

                @pl.when(r2 + 2 < CHUNK)
                def _():
                    gather_start(r2 + 2, rows0, sem0)

                gather_wait(rows1, sem1)
                compute(r2 + 1, rows1)
                return carry

            lax.fori_loop(0, CHUNK // 2, pair_body, 0)
            pltpu.sync_copy(out_v, out_hbm.at[ci].at[pl.ds(base, CHUNK)])
            return carry

        lax.fori_loop(0, N_CHUNKS, chunk_body, 0)

    return k(table, vals, cols)


def _add_tc(p0, p1):
    def body(a_ref, b_ref, o_ref):
        o_ref[...] = a_ref[...] + b_ref[...]

    grid = (N_ROWS // 1024,)
    spec = pl.BlockSpec((1024, BATCH), lambda i: (i, 0))
    return pl.pallas_call(
        body,
        out_shape=jax.ShapeDtypeStruct((N_ROWS, BATCH), jnp.float32),
        grid=grid,
        in_specs=[spec, spec],
        out_specs=spec,
    )(p0, p1)


def _interleave_perm():
    p = np.empty((BATCH,), np.int32)
    for k in range(BATCH // 32):
        for t in range(16):
            p[32 * k + 2 * t] = 32 * k + t
            p[32 * k + 2 * t + 1] = 32 * k + 16 + t
    return p


_PERM = _interleave_perm()


@jax.jit
def kernel(x, c_0, c_1, c_2):
    del c_2
    tb = x[jnp.asarray(_PERM)].T.astype(jnp.bfloat16)
    table = lax.bitcast_convert_type(
        tb.reshape(N_COLS, BATCH // 2, 2), jnp.int32)
    v = c_0.reshape(N_ROWS, K)
    c = c_1.reshape(N_ROWS, K)
    in_hi = c >= HALF_COLS
    vals2 = jnp.zeros((NC, N_ROWS, KP), jnp.float32)
    vals2 = vals2.at[0, :, :K].set(jnp.where(in_hi, 0.0, v))
    vals2 = vals2.at[1, :, :K].set(jnp.where(in_hi, v, 0.0))
    cols2 = jnp.zeros((NC, N_ROWS, KP), jnp.int32)
    cols2 = cols2.at[0, :, :K].set(jnp.where(in_hi, 0, c))
    cols2 = cols2.at[1, :, :K].set(jnp.where(in_hi, c - HALF_COLS, 0))
    partials = _spmv_sc(table, vals2, cols2)
    yT = _add_tc(partials[0], partials[1])
    return yT.T


# speedup vs baseline: 2.2763x; 1.3805x over previous
import functools

import jax
import jax.numpy as jnp
import numpy as np
from jax import lax
from jax.experimental import pallas as pl
from jax.experimental.pallas import tpu as pltpu
from jax.experimental.pallas import tpu_sc as plsc

N_ROWS = 16384
N_COLS = 16384
K = 163
KP = 176
KPC = 192
HALF_COLS = N_COLS // 2
BATCH = 256
NC = 2
NS = 16
ROWS_PER_W = N_ROWS // NS
LANES = 16
VB = BATCH // LANES
CHUNK = 8
N_CHUNKS = ROWS_PER_W // CHUNK
LOAD_ROWS = HALF_COLS // NS
LOAD_STEP = 128


def _spmv_sc(table, vals, cols):
    mesh = plsc.VectorSubcoreMesh(core_axis_name="c", subcore_axis_name="s")

    @functools.partial(
        pl.kernel,
        mesh=mesh,
        out_type=jax.ShapeDtypeStruct((NC, N_ROWS, BATCH), jnp.float32),
        compiler_params=pltpu.CompilerParams(use_tc_tiling_on_sc=False,
                                             needs_layout_passes=False),
        scratch_types=[
            pltpu.VMEM_SHARED((HALF_COLS, BATCH // 2), jnp.int32),
            pltpu.VMEM((CHUNK, KP), jnp.int32),
            pltpu.VMEM((CHUNK, KP), jnp.float32),
            pltpu.VMEM((CHUNK, KPC), jnp.int32),
            pltpu.VMEM((CHUNK, KPC), jnp.float32),
            pltpu.VMEM((KP, BATCH // 2), jnp.int32),
            pltpu.VMEM((KP, BATCH // 2), jnp.int32),
            pltpu.VMEM((CHUNK, BATCH), jnp.float32),
            pltpu.SemaphoreType.DMA,
            pltpu.SemaphoreType.DMA,
        ],
    )
    def k(table_hbm, vals_hbm, cols_hbm, out_hbm, table_sp, colbuf, valbuf,
          idx_c, vals_c, rows0, rows1, out_v, sem0, sem1):
        ci = lax.axis_index("c")
        si = lax.axis_index("s")

        def stage_body(b, carry):
            src_row = ci * HALF_COLS + si * LOAD_ROWS + b * LOAD_STEP
            dst_row = si * LOAD_ROWS + b * LOAD_STEP
            pltpu.sync_copy(table_hbm.at[pl.ds(src_row, LOAD_STEP)],
                            rows0.at[pl.ds(0, LOAD_STEP)])
            pltpu.sync_copy(rows0.at[pl.ds(0, LOAD_STEP)],
                            table_sp.at[pl.ds(dst_row, LOAD_STEP)])
            return carry

        lax.fori_loop(0, LOAD_ROWS // LOAD_STEP, stage_body, 0)
        plsc.subcore_barrier()

        row0 = si * ROWS_PER_W

        def gather_start(r, dst, sem, nbc):
            def gb(q, carry):
                iv = idx_c[r, pl.ds(q * LANES, LANES)]
                pltpu.async_copy(
                    table_sp.at[iv],
                    dst.at[pl.ds(q * LANES, LANES)], sem)
                return carry

            lax.fori_loop(0, nbc, gb, 0)

        def gather_wait(dst, sem, nbc):
            zv = jnp.zeros((LANES,), jnp.int32)

            def gw(q, carry):
                pltpu.make_async_copy(
                    table_sp.at[zv],
                    dst.at[pl.ds(q * LANES, LANES)], sem).wait()
                return carry

            lax.fori_loop(0, nbc, gw, 0)

        def compute(r, src, nbc):
            HB = VB // 2

            for h in range(2):
                def j_body(jb, accs, h=h):
                    j0 = pl.multiple_of(jb * LANES, LANES)
                    vblock = vals_c[r, pl.ds(j0, LANES)]
                    for t in range(LANES):
                        vv = jnp.full((LANES,), vblock[t], jnp.float32)
                        new = list(accs)
                        for u in range(HB // 2):
                            ld = src[j0 + t,
                                     pl.ds(h * (HB * LANES) // 2 + u * LANES,
                                           LANES)]
                            ab = plsc.bitcast(ld, jnp.bfloat16)
                            a, b = plsc.unpack(
                                ab, format=plsc.PackFormat.INTERLEAVED)
                            new[2 * u] = new[2 * u] + vv * a
                            new[2 * u + 1] = new[2 * u + 1] + vv * b
                        accs = tuple(new)
                    return accs

                accs = lax.fori_loop(
                    0, nbc, j_body,
                    tuple(jnp.zeros((LANES,), jnp.float32)
                          for _ in range(HB)))
                for k2 in range(HB):
                    out_v[r, pl.ds((h * HB + k2) * LANES, LANES)] = accs[k2]

        zvi = jnp.zeros((LANES,), jnp.int32)
        zvf = jnp.zeros((LANES,), jnp.float32)

        for r0i in range(CHUNK):
            for z in range(KPC // LANES):
                idx_c[r0i, pl.ds(z * LANES, LANES)] = zvi

        def compact_row(r, maxnb):
            for z in range(KPC // LANES):
                vals_c[r, pl.ds(z * LANES, LANES)] = zvf

            def blk(jb, cnt):
                j0 = pl.multiple_of(jb * LANES, LANES)
                cv = colbuf[r, pl.ds(j0, LANES)]
                vv = valbuf[r, pl.ds(j0, LANES)]
                m = lax.shift_right_arithmetic(cv, 13) == ci
                lidx = lax.bitwise_and(cv, HALF_COLS - 1)
                plsc.store_compressed(idx_c.at[r, pl.ds(cnt, LANES)],
                                      lidx, mask=m)
                plsc.store_compressed(vals_c.at[r, pl.ds(cnt, LANES)],
                                      vv, mask=m)
                pc = plsc.all_reduce_population_count(m)
                return cnt + pc[0]

            cnt = lax.fori_loop(0, KP // LANES, blk, jnp.int32(0))
            nb = (cnt + (LANES - 1)) // LANES
            return jnp.maximum(maxnb, nb)

        def chunk_body(c, carry):
            base = row0 + c * CHUNK
            pltpu.sync_copy(cols_hbm.at[pl.ds(base, CHUNK)], colbuf)
            pltpu.sync_copy(vals_hbm.at[pl.ds(base, CHUNK)], valbuf)
            nbc = lax.fori_loop(0, CHUNK, compact_row, jnp.int32(0))
            gather_start(0, rows0, sem0, nbc)

            def pair_body(i, carry):
                nbc = carry
                r2 = i * 2
                gather_start(r2 + 1, rows1, sem1, nbc)
                gather_wait(rows0, sem0, nbc)
                compute(r2, rows0, nbc)

                @pl.when(r2 + 2 < CHUNK)
                def _():
                    gather_start(r2 + 2, rows0, sem0, nbc)

                gather_wait(rows1, sem1, nbc)
                compute(r2 + 1, rows1, nbc)
                return carry

            lax.fori_loop(0, CHUNK // 2, pair_body, nbc)
            pltpu.sync_copy(out_v, out_hbm.at[ci].at[pl.ds(base, CHUNK)])
            return carry

        lax.fori_loop(0, N_CHUNKS, chunk_body, 0)

    return k(table, vals, cols)


def _add_tc(p0, p1):
    def body(a_ref, b_ref, o_ref):
        o_ref[...] = a_ref[...] + b_ref[...]

    grid = (N_ROWS // 1024,)
    spec = pl.BlockSpec((1024, BATCH), lambda i: (i, 0))
    return pl.pallas_call(
        body,
        out_shape=jax.ShapeDtypeStruct((N_ROWS, BATCH), jnp.float32),
        grid=grid,
        in_specs=[spec, spec],
        out_specs=spec,
    )(p0, p1)


def _interleave_perm():
    p = np.empty((BATCH,), np.int32)
    for k in range(BATCH // 32):
        for t in range(16):
            p[32 * k + 2 * t] = 32 * k + t
            p[32 * k + 2 * t + 1] = 32 * k + 16 + t
    return p


_PERM = _interleave_perm()


@jax.jit
def kernel(x, c_0, c_1, c_2):
    del c_2
    tb = x[jnp.asarray(_PERM)].T.astype(jnp.bfloat16)
    table = lax.bitcast_convert_type(
        tb.reshape(N_COLS, BATCH // 2, 2), jnp.int32)
    vals2 = jnp.zeros((N_ROWS, KP), jnp.float32)
    vals2 = vals2.at[:, :K].set(c_0.reshape(N_ROWS, K))
    cols2 = jnp.full((N_ROWS, KP), -1, jnp.int32)
    cols2 = cols2.at[:, :K].set(c_1.reshape(N_ROWS, K))
    partials = _spmv_sc(table, vals2, cols2)
    yT = _add_tc(partials[0], partials[1])
    return yT.T
